# baseline (speedup 1.0000x reference)
import jax
import jax.numpy as jnp
from jax import lax
from jax.experimental import pallas as pl
from jax.experimental.pallas import tpu as pltpu
from jax.experimental.pallas import tpu_sc as plsc

_D = 128
_B = 4096
_S = 200
_LANES = 16
_NW = 32
_RPW = _B // _NW
_C0 = 128
_C1 = _S - _C0


def _body(ids_hbm, langids_hbm, tok_hbm, lang_hbm, out_hbm,
          idx_v, langids_v, langrows_v, rows0, rows1,
          gsem0, gsem1, osem0, osem1, seml):
    nc = 2
    wid = lax.axis_index("c") * (_NW // nc) + lax.axis_index("s")
    row0 = wid * _RPW
    rows_v = (rows0, rows1)
    gsem = (gsem0, gsem1)
    osem = (osem0, osem1)

    pltpu.sync_copy(ids_hbm.at[pl.ds(row0 * _S, _RPW * _S)], idx_v)
    pltpu.sync_copy(langids_hbm.at[pl.ds(row0, _RPW)], langids_v)
    pltpu.async_copy(lang_hbm.at[langids_v], langrows_v, seml).wait()

    def fill(i, b):
        lv = [langrows_v[i, pl.ds(l * _LANES, _LANES)]
              for l in range(_D // _LANES)]

        def one(r, _):
            for l in range(_D // _LANES):
                rows_v[b][r, pl.ds(l * _LANES, _LANES)] = lv[l]
            return 0

        lax.fori_loop(0, _S, one, 0)

    def gstart(i, b):
        base = i * _S
        pltpu.async_copy(tok_hbm.at[idx_v.at[pl.ds(base, _C0)]],
                         rows_v[b].at[pl.ds(0, _C0)], gsem[b], add=True)
        pltpu.async_copy(tok_hbm.at[idx_v.at[pl.ds(base + _C0, _C1)]],
                         rows_v[b].at[pl.ds(_C0, _C1)], gsem[b], add=True)

    def gwait(b):
        pltpu.make_async_copy(tok_hbm.at[pl.ds(0, _C0)],
                              rows_v[b].at[pl.ds(0, _C0)], gsem[b]).wait()
        pltpu.make_async_copy(tok_hbm.at[pl.ds(0, _C1)],
                              rows_v[b].at[pl.ds(_C0, _C1)], gsem[b]).wait()

    def ostart(i, b):
        pltpu.async_copy(rows_v[b], out_hbm.at[pl.ds((row0 + i) * _S, _S)],
                         osem[b])

    def owait(b):
        pltpu.make_async_copy(rows_v[b], out_hbm.at[pl.ds(0, _S)],
                              osem[b]).wait()

    def stage(i, b):
        fill(i, b)
        gstart(i, b)

    stage(0, 0)

    def outer(g, _):
        a = 2 * g
        lax.cond(g > 0, lambda: owait(1), lambda: None)
        stage(a + 1, 1)
        gwait(0)
        ostart(a, 0)
        def stage_next():
            owait(0)
            stage(a + 2, 0)
        lax.cond(g < _RPW // 2 - 1, stage_next, lambda: None)
        gwait(1)
        ostart(a + 1, 1)
        return 0

    lax.fori_loop(0, _RPW // 2, outer, 0)
    owait(0)
    owait(1)


@jax.jit
def _run(ids_flat, language_ids, token_table, language_table):
    mesh = plsc.VectorSubcoreMesh(core_axis_name="c", subcore_axis_name="s")
    fn = pl.kernel(
        _body,
        out_type=jax.ShapeDtypeStruct((_B * _S, _D), jnp.float32),
        mesh=mesh,
        scratch_types=[
            pltpu.VMEM((_RPW * _S,), jnp.int32),
            pltpu.VMEM((_RPW,), jnp.int32),
            pltpu.VMEM((_RPW, _D), jnp.float32),
            pltpu.VMEM((_S, _D), jnp.float32),
            pltpu.VMEM((_S, _D), jnp.float32),
            pltpu.SemaphoreType.DMA,
            pltpu.SemaphoreType.DMA,
            pltpu.SemaphoreType.DMA,
            pltpu.SemaphoreType.DMA,
            pltpu.SemaphoreType.DMA,
        ],
    )
    return fn(ids_flat, language_ids, token_table, language_table)


def kernel(input_ids, language_ids, token_table, language_table):
    ids_flat = input_ids.reshape(-1).astype(jnp.int32)
    lang_ids = language_ids.astype(jnp.int32)
    out = _run(ids_flat, lang_ids, token_table, language_table)
    return out.reshape(_B, _S, _D)
